# baseline (speedup 1.0000x reference)
import jax
import jax.numpy as jnp
from jax import lax
from jax.experimental import pallas as pl
from jax.experimental.pallas import tpu as pltpu
from jax.experimental.pallas import tpu_sc as plsc

_B = 16384
_D = 16
_NC, _NS = 2, 16
_NW = _NC * _NS
_BPW = _B // _NW
_N = 1000000
_NCOLS = 7813
_TAIL = _NCOLS - 1
_Q = 4
_NQ = (_NCOLS + _Q - 1) // _Q
_TAILQ = _NQ - 1
_SLOT = _D * _Q * 128
_L = 16
_CHUNKS = _BPW // _L

_NBUF = 8
_LAG = 4


def _detile_body(zt_hbm, ztail_hbm, zst_hbm, bufs, in_sem, out_sem):
    wid = lax.axis_index("s") * _NC + lax.axis_index("c")
    nit = (_NQ - wid + _NW - 1) // _NW

    def in_copy(g):
        q = wid + g * _NW
        b = lax.rem(g, _NBUF)

        @pl.when(q != _TAILQ)
        def _():
            pltpu.make_async_copy(
                zt_hbm.at[:, pl.ds(q * (_Q * 128), _Q * 128)],
                bufs.at[b],
                in_sem,
            ).start()

        @pl.when(q == _TAILQ)
        def _():
            pltpu.make_async_copy(
                ztail_hbm, bufs.at[b, :, pl.ds(0, 128)], in_sem
            ).start()

    def in_wait(g):
        q = wid + g * _NW
        b = lax.rem(g, _NBUF)

        @pl.when(q != _TAILQ)
        def _():
            pltpu.make_async_copy(
                zt_hbm.at[:, pl.ds(0, _Q * 128)], bufs.at[b], in_sem
            ).wait()

        @pl.when(q == _TAILQ)
        def _():
            pltpu.make_async_copy(
                ztail_hbm, bufs.at[b, :, pl.ds(0, 128)], in_sem
            ).wait()

    def out_copy(g):
        q = wid + g * _NW
        pltpu.make_async_copy(
            bufs.at[lax.rem(g, _NBUF)], zst_hbm.at[q], out_sem
        ).start()

    def out_wait():
        pltpu.make_async_copy(
            bufs.at[0], zst_hbm.at[wid], out_sem
        ).wait()

    def step(g, carry):
        @pl.when(g >= _NBUF)
        def _():
            out_wait()

        @pl.when(g < nit)
        def _():
            in_copy(g)

        @pl.when(jnp.logical_and(g >= _LAG, g - _LAG < nit))
        def _():
            in_wait(g - _LAG)
            out_copy(g - _LAG)

        return carry

    lax.fori_loop(0, nit + _LAG, step, 0, unroll=False)

    def drain(k, carry):
        @pl.when(k < jnp.minimum(nit, _NBUF - _LAG))
        def _():
            out_wait()

        return carry

    lax.fori_loop(0, _NBUF - _LAG, drain, 0, unroll=False)


def _gather_body(zst_hbm, idx_hbm, out_hbm, idx_v, off_v, val_v, isem, gsem):
    wid = lax.axis_index("s") * _NC + lax.axis_index("c")
    base = wid * _BPW
    cp = pltpu.make_async_copy(idx_hbm.at[pl.ds(base, _BPW)], idx_v, isem)
    cp.start()
    cp.wait()

    def compute(k, carry):
        idx = idx_v[pl.ds(k * _L, _L)]
        c = lax.shift_right_logical(idx, 7)
        il = lax.bitwise_and(idx, 127)
        s = lax.shift_right_logical(c, 2)
        cl = lax.bitwise_and(c, 3)
        boff = s * _SLOT + cl * 128 + il
        for j in range(_D):
            off_v[j, pl.ds(k * _L, _L)] = boff + (j * (_Q * 128))
        return carry

    lax.fori_loop(0, _CHUNKS, compute, 0, unroll=False)

    for j in range(_D):
        pltpu.make_async_copy(
            zst_hbm.at[off_v.at[j]], val_v.at[j], gsem
        ).start()
    for j in range(_D):
        pltpu.make_async_copy(
            zst_hbm.at[off_v.at[j]], val_v.at[j], gsem
        ).wait()

    for j in range(_D):
        pltpu.sync_copy(val_v.at[j], out_hbm.at[j, pl.ds(base, _BPW)])


def kernel(indices, z):
    mesh = plsc.VectorSubcoreMesh(core_axis_name="c", subcore_axis_name="s")
    detile = pl.kernel(
        _detile_body,
        mesh=mesh,
        out_type=jax.ShapeDtypeStruct((_NQ, _D, _Q * 128), jnp.float32),
        scratch_types=[
            pltpu.VMEM((_NBUF, _D, _Q * 128), jnp.float32),
            pltpu.SemaphoreType.DMA,
            pltpu.SemaphoreType.DMA,
        ],
    )
    gather = pl.kernel(
        _gather_body,
        mesh=mesh,
        out_type=jax.ShapeDtypeStruct((_D, _B), jnp.float32),
        scratch_types=[
            pltpu.VMEM((_BPW,), jnp.int32),
            pltpu.VMEM((_D, _BPW), jnp.int32),
            pltpu.VMEM((_D, _BPW), jnp.float32),
            pltpu.SemaphoreType.DMA,
            pltpu.SemaphoreType.DMA,
        ],
        compiler_params=pltpu.CompilerParams(use_tc_tiling_on_sc=False),
    )
    zt = z.T
    ztail = jnp.pad(zt[:, _TAIL * 128 :], ((0, 0), (0, _NCOLS * 128 - _N)))
    zst = detile(zt, ztail)
    zst_flat = zst.reshape(-1)
    out_t = gather(zst_flat, indices.astype(jnp.int32))
    return out_t.T


# speedup vs baseline: 1.5866x; 1.5866x over previous
import jax
import jax.numpy as jnp
from jax import lax
from jax.experimental import pallas as pl
from jax.experimental.pallas import tpu as pltpu
from jax.experimental.pallas import tpu_sc as plsc

_B = 16384
_D = 16
_NC, _NS = 2, 16
_NW = _NC * _NS
_BPW = _B // _NW
_N = 1000000
_NCOLS = 7813
_TAIL = _NCOLS - 1
_Q = 1
_QSHIFT = _Q.bit_length() - 1
_NQ = (_NCOLS + _Q - 1) // _Q
_TAILQ = _NQ - 1
_SLOT = _D * _Q * 128
_L = 16
_CHUNKS = _BPW // _L

_NBUF = 16
_LAG = 8


def _detile_body(zt_hbm, ztail_hbm, zst_hbm, bufs, in_sem, out_sem):
    wid = lax.axis_index("s") * _NC + lax.axis_index("c")
    nit = (_NQ - wid + _NW - 1) // _NW

    def in_copy(g):
        q = wid + g * _NW
        b = lax.rem(g, _NBUF)

        @pl.when(q != _TAILQ)
        def _():
            pltpu.make_async_copy(
                zt_hbm.at[:, pl.ds(q * (_Q * 128), _Q * 128)],
                bufs.at[b],
                in_sem,
            ).start()

        @pl.when(q == _TAILQ)
        def _():
            pltpu.make_async_copy(
                ztail_hbm, bufs.at[b, :, pl.ds(0, 128)], in_sem
            ).start()

    def in_wait(g):
        q = wid + g * _NW
        b = lax.rem(g, _NBUF)

        @pl.when(q != _TAILQ)
        def _():
            pltpu.make_async_copy(
                zt_hbm.at[:, pl.ds(0, _Q * 128)], bufs.at[b], in_sem
            ).wait()

        @pl.when(q == _TAILQ)
        def _():
            pltpu.make_async_copy(
                ztail_hbm, bufs.at[b, :, pl.ds(0, 128)], in_sem
            ).wait()

    def out_copy(g):
        q = wid + g * _NW
        pltpu.make_async_copy(
            bufs.at[lax.rem(g, _NBUF)], zst_hbm.at[q], out_sem
        ).start()

    def out_wait():
        pltpu.make_async_copy(
            bufs.at[0], zst_hbm.at[wid], out_sem
        ).wait()

    def step(g, carry):
        @pl.when(g >= _NBUF)
        def _():
            out_wait()

        @pl.when(g < nit)
        def _():
            in_copy(g)

        @pl.when(jnp.logical_and(g >= _LAG, g - _LAG < nit))
        def _():
            in_wait(g - _LAG)
            out_copy(g - _LAG)

        return carry

    lax.fori_loop(0, nit + _LAG, step, 0, unroll=False)

    def drain(k, carry):
        @pl.when(k < jnp.minimum(nit, _NBUF - _LAG))
        def _():
            out_wait()

        return carry

    lax.fori_loop(0, _NBUF - _LAG, drain, 0, unroll=False)


def _gather_body(zst_hbm, idx_hbm, out_hbm, idx_v, off_v, val_v, isem, gsem):
    wid = lax.axis_index("s") * _NC + lax.axis_index("c")
    base = wid * _BPW
    cp = pltpu.make_async_copy(idx_hbm.at[pl.ds(base, _BPW)], idx_v, isem)
    cp.start()
    cp.wait()

    def compute(k, carry):
        idx = idx_v[pl.ds(k * _L, _L)]
        c = lax.shift_right_logical(idx, 7)
        il = lax.bitwise_and(idx, 127)
        s = lax.shift_right_logical(c, _QSHIFT)
        cl = lax.bitwise_and(c, _Q - 1)
        boff = s * _SLOT + cl * 128 + il
        for j in range(_D):
            off_v[j, pl.ds(k * _L, _L)] = boff + (j * (_Q * 128))
        return carry

    lax.fori_loop(0, _CHUNKS, compute, 0, unroll=False)

    for j in range(_D):
        pltpu.make_async_copy(
            zst_hbm.at[off_v.at[j]], val_v.at[j], gsem
        ).start()
    for j in range(_D):
        pltpu.make_async_copy(
            zst_hbm.at[off_v.at[j]], val_v.at[j], gsem
        ).wait()

    for j in range(_D):
        pltpu.sync_copy(val_v.at[j], out_hbm.at[j, pl.ds(base, _BPW)])


def kernel(indices, z):
    mesh = plsc.VectorSubcoreMesh(core_axis_name="c", subcore_axis_name="s")
    detile = pl.kernel(
        _detile_body,
        mesh=mesh,
        out_type=jax.ShapeDtypeStruct((_NQ, _D, _Q * 128), jnp.float32),
        scratch_types=[
            pltpu.VMEM((_NBUF, _D, _Q * 128), jnp.float32),
            pltpu.SemaphoreType.DMA,
            pltpu.SemaphoreType.DMA,
        ],
    )
    gather = pl.kernel(
        _gather_body,
        mesh=mesh,
        out_type=jax.ShapeDtypeStruct((_D, _B), jnp.float32),
        scratch_types=[
            pltpu.VMEM((_BPW,), jnp.int32),
            pltpu.VMEM((_D, _BPW), jnp.int32),
            pltpu.VMEM((_D, _BPW), jnp.float32),
            pltpu.SemaphoreType.DMA,
            pltpu.SemaphoreType.DMA,
        ],
        compiler_params=pltpu.CompilerParams(use_tc_tiling_on_sc=False),
    )
    zt = z.T
    ztail = jnp.pad(zt[:, _TAIL * 128 :], ((0, 0), (0, _NCOLS * 128 - _N)))
    zst = detile(zt, ztail)
    zst_flat = zst.reshape(-1)
    out_t = gather(zst_flat, indices.astype(jnp.int32))
    return out_t.T


# speedup vs baseline: 1.6073x; 1.0131x over previous
import jax
import jax.numpy as jnp
from jax import lax
from jax.experimental import pallas as pl
from jax.experimental.pallas import tpu as pltpu
from jax.experimental.pallas import tpu_sc as plsc

_B = 16384
_D = 16
_NC, _NS = 2, 16
_NW = _NC * _NS
_BPW = _B // _NW
_N = 1000000
_NCOLS = 7813
_TAIL = _NCOLS - 1
_Q = 1
_QSHIFT = _Q.bit_length() - 1
_NQ = (_NCOLS + _Q - 1) // _Q
_TAILQ = _NQ - 1
_SLOT = _D * _Q * 128
_L = 16
_CHUNKS = _BPW // _L

_NBUF = 24
_LAG = 12


def _detile_body(zt_hbm, ztail_hbm, zst_hbm, bufs, in_sem, out_sem):
    wid = lax.axis_index("s") * _NC + lax.axis_index("c")
    nit = (_NQ - wid + _NW - 1) // _NW

    def in_copy(g):
        q = wid + g * _NW
        b = lax.rem(g, _NBUF)

        @pl.when(q != _TAILQ)
        def _():
            pltpu.make_async_copy(
                zt_hbm.at[:, pl.ds(q * (_Q * 128), _Q * 128)],
                bufs.at[b],
                in_sem,
            ).start()

        @pl.when(q == _TAILQ)
        def _():
            pltpu.make_async_copy(
                ztail_hbm, bufs.at[b, :, pl.ds(0, 128)], in_sem
            ).start()

    def in_wait(g):
        q = wid + g * _NW
        b = lax.rem(g, _NBUF)

        @pl.when(q != _TAILQ)
        def _():
            pltpu.make_async_copy(
                zt_hbm.at[:, pl.ds(0, _Q * 128)], bufs.at[b], in_sem
            ).wait()

        @pl.when(q == _TAILQ)
        def _():
            pltpu.make_async_copy(
                ztail_hbm, bufs.at[b, :, pl.ds(0, 128)], in_sem
            ).wait()

    def out_copy(g):
        q = wid + g * _NW
        pltpu.make_async_copy(
            bufs.at[lax.rem(g, _NBUF)], zst_hbm.at[q], out_sem
        ).start()

    def out_wait():
        pltpu.make_async_copy(
            bufs.at[0], zst_hbm.at[wid], out_sem
        ).wait()

    def step(g, carry):
        @pl.when(g >= _NBUF)
        def _():
            out_wait()

        @pl.when(g < nit)
        def _():
            in_copy(g)

        @pl.when(jnp.logical_and(g >= _LAG, g - _LAG < nit))
        def _():
            in_wait(g - _LAG)
            out_copy(g - _LAG)

        return carry

    lax.fori_loop(0, nit + _LAG, step, 0, unroll=False)

    def drain(k, carry):
        @pl.when(k < jnp.minimum(nit, _NBUF - _LAG))
        def _():
            out_wait()

        return carry

    lax.fori_loop(0, _NBUF - _LAG, drain, 0, unroll=False)


def _gather_body(zst_hbm, idx_hbm, out_hbm, idx_v, off_v, val_v, isem, gsem):
    wid = lax.axis_index("s") * _NC + lax.axis_index("c")
    base = wid * _BPW
    cp = pltpu.make_async_copy(idx_hbm.at[pl.ds(base, _BPW)], idx_v, isem)
    cp.start()
    cp.wait()

    def compute(k, carry):
        idx = idx_v[pl.ds(k * _L, _L)]
        c = lax.shift_right_logical(idx, 7)
        il = lax.bitwise_and(idx, 127)
        s = lax.shift_right_logical(c, _QSHIFT)
        cl = lax.bitwise_and(c, _Q - 1)
        boff = s * _SLOT + cl * 128 + il
        for j in range(_D):
            off_v[j, pl.ds(k * _L, _L)] = boff + (j * (_Q * 128))
        return carry

    lax.fori_loop(0, _CHUNKS, compute, 0, unroll=False)

    for j in range(_D):
        pltpu.make_async_copy(
            zst_hbm.at[off_v.at[j]], val_v.at[j], gsem
        ).start()
    for j in range(_D):
        pltpu.make_async_copy(
            zst_hbm.at[off_v.at[j]], val_v.at[j], gsem
        ).wait()

    for j in range(_D):
        pltpu.sync_copy(val_v.at[j], out_hbm.at[j, pl.ds(base, _BPW)])


def kernel(indices, z):
    mesh = plsc.VectorSubcoreMesh(core_axis_name="c", subcore_axis_name="s")
    detile = pl.kernel(
        _detile_body,
        mesh=mesh,
        out_type=jax.ShapeDtypeStruct((_NQ, _D, _Q * 128), jnp.float32),
        scratch_types=[
            pltpu.VMEM((_NBUF, _D, _Q * 128), jnp.float32),
            pltpu.SemaphoreType.DMA,
            pltpu.SemaphoreType.DMA,
        ],
    )
    gather = pl.kernel(
        _gather_body,
        mesh=mesh,
        out_type=jax.ShapeDtypeStruct((_D, _B), jnp.float32),
        scratch_types=[
            pltpu.VMEM((_BPW,), jnp.int32),
            pltpu.VMEM((_D, _BPW), jnp.int32),
            pltpu.VMEM((_D, _BPW), jnp.float32),
            pltpu.SemaphoreType.DMA,
            pltpu.SemaphoreType.DMA,
        ],
        compiler_params=pltpu.CompilerParams(use_tc_tiling_on_sc=False),
    )
    zt = z.T
    ztail = jnp.pad(zt[:, _TAIL * 128 :], ((0, 0), (0, _NCOLS * 128 - _N)))
    zst = detile(zt, ztail)
    zst_flat = zst.reshape(-1)
    out_t = gather(zst_flat, indices.astype(jnp.int32))
    return out_t.T


# speedup vs baseline: 1.6326x; 1.0158x over previous
import jax
import jax.numpy as jnp
from jax import lax
from jax.experimental import pallas as pl
from jax.experimental.pallas import tpu as pltpu
from jax.experimental.pallas import tpu_sc as plsc

_B = 16384
_D = 16
_NC, _NS = 2, 16
_NW = _NC * _NS
_BPW = _B // _NW
_N = 1000000
_NCOLS = 7813
_TAIL = _NCOLS - 1
_Q = 1
_QSHIFT = _Q.bit_length() - 1
_NQ = (_NCOLS + _Q - 1) // _Q
_TAILQ = _NQ - 1
_SLOT = _D * _Q * 128
_L = 16
_CHUNKS = _BPW // _L

_NBUF = 24
_LAG = 12


def _detile_body(zt_hbm, ztail_hbm, zst_hbm, bufs, in_sem, out_sem):
    wid = lax.axis_index("s") * _NC + lax.axis_index("c")
    nit = (_NQ - wid + _NW - 1) // _NW

    def in_copy(g):
        q = wid + g * _NW
        b = lax.rem(g, _NBUF)

        @pl.when(q != _TAILQ)
        def _():
            pltpu.make_async_copy(
                zt_hbm.at[:, pl.ds(q * (_Q * 128), _Q * 128)],
                bufs.at[b],
                in_sem,
            ).start()

        @pl.when(q == _TAILQ)
        def _():
            pltpu.make_async_copy(
                ztail_hbm, bufs.at[b, :, pl.ds(0, 128)], in_sem
            ).start()

    def in_wait(g):
        q = wid + g * _NW
        b = lax.rem(g, _NBUF)

        @pl.when(q != _TAILQ)
        def _():
            pltpu.make_async_copy(
                zt_hbm.at[:, pl.ds(0, _Q * 128)], bufs.at[b], in_sem
            ).wait()

        @pl.when(q == _TAILQ)
        def _():
            pltpu.make_async_copy(
                ztail_hbm, bufs.at[b, :, pl.ds(0, 128)], in_sem
            ).wait()

    def out_copy(g):
        q = wid + g * _NW
        pltpu.make_async_copy(
            bufs.at[lax.rem(g, _NBUF)], zst_hbm.at[q], out_sem
        ).start()

    def out_wait():
        pltpu.make_async_copy(
            bufs.at[0], zst_hbm.at[wid], out_sem
        ).wait()

    def step(g, carry):
        @pl.when(g >= _NBUF)
        def _():
            out_wait()

        @pl.when(g < nit)
        def _():
            in_copy(g)

        @pl.when(jnp.logical_and(g >= _LAG, g - _LAG < nit))
        def _():
            in_wait(g - _LAG)
            out_copy(g - _LAG)

        return carry

    lax.fori_loop(0, nit + _LAG, step, 0, unroll=False)

    def drain(k, carry):
        @pl.when(k < jnp.minimum(nit, _NBUF - _LAG))
        def _():
            out_wait()

        return carry

    lax.fori_loop(0, _NBUF - _LAG, drain, 0, unroll=False)


def _gather_body(
    zst_hbm, idx_hbm, out_hbm, idx_v, off_v, val_v, isem, gsem, osem
):
    wid = lax.axis_index("s") * _NC + lax.axis_index("c")
    base = wid * _BPW
    cp = pltpu.make_async_copy(idx_hbm.at[pl.ds(base, _BPW)], idx_v, isem)
    cp.start()
    cp.wait()

    def compute(k, carry):
        idx = idx_v[pl.ds(k * _L, _L)]
        c = lax.shift_right_logical(idx, 7)
        il = lax.bitwise_and(idx, 127)
        s = lax.shift_right_logical(c, _QSHIFT)
        cl = lax.bitwise_and(c, _Q - 1)
        boff = s * _SLOT + cl * 128 + il
        for j in range(_D):
            off_v[j, pl.ds(k * _L, _L)] = boff + (j * (_Q * 128))
        return carry

    half = _BPW // 2

    def fire(j, lo):
        pltpu.make_async_copy(
            zst_hbm.at[off_v.at[j, pl.ds(lo, half)]],
            val_v.at[j, pl.ds(lo, half)],
            gsem,
        ).start()

    def drain(j, lo):
        pltpu.make_async_copy(
            zst_hbm.at[off_v.at[j, pl.ds(lo, half)]],
            val_v.at[j, pl.ds(lo, half)],
            gsem,
        ).wait()

    lax.fori_loop(0, _CHUNKS // 2, compute, 0, unroll=False)
    for j in range(_D):
        fire(j, 0)
    lax.fori_loop(_CHUNKS // 2, _CHUNKS, compute, 0, unroll=False)
    for j in range(_D):
        fire(j, half)
    for j in range(_D):
        drain(j, 0)
        pltpu.make_async_copy(
            val_v.at[j, pl.ds(0, half)],
            out_hbm.at[j, pl.ds(base, half)],
            osem,
        ).start()
    for j in range(_D):
        drain(j, half)
        pltpu.make_async_copy(
            val_v.at[j, pl.ds(half, half)],
            out_hbm.at[j, pl.ds(base + half, half)],
            osem,
        ).start()
    for j in range(2 * _D):
        pltpu.make_async_copy(
            val_v.at[0, pl.ds(0, half)],
            out_hbm.at[0, pl.ds(base, half)],
            osem,
        ).wait()


def kernel(indices, z):
    mesh = plsc.VectorSubcoreMesh(core_axis_name="c", subcore_axis_name="s")
    detile = pl.kernel(
        _detile_body,
        mesh=mesh,
        out_type=jax.ShapeDtypeStruct((_NQ, _D, _Q * 128), jnp.float32),
        scratch_types=[
            pltpu.VMEM((_NBUF, _D, _Q * 128), jnp.float32),
            pltpu.SemaphoreType.DMA,
            pltpu.SemaphoreType.DMA,
        ],
    )
    gather = pl.kernel(
        _gather_body,
        mesh=mesh,
        out_type=jax.ShapeDtypeStruct((_D, _B), jnp.float32),
        scratch_types=[
            pltpu.VMEM((_BPW,), jnp.int32),
            pltpu.VMEM((_D, _BPW), jnp.int32),
            pltpu.VMEM((_D, _BPW), jnp.float32),
            pltpu.SemaphoreType.DMA,
            pltpu.SemaphoreType.DMA,
            pltpu.SemaphoreType.DMA,
        ],
        compiler_params=pltpu.CompilerParams(use_tc_tiling_on_sc=False),
    )
    zt = z.T
    ztail = jnp.pad(zt[:, _TAIL * 128 :], ((0, 0), (0, _NCOLS * 128 - _N)))
    zst = detile(zt, ztail)
    zst_flat = zst.reshape(-1)
    out_t = gather(zst_flat, indices.astype(jnp.int32))
    return out_t.T
